# baseline (speedup 1.0000x reference)
import functools

import jax
import jax.numpy as jnp
from jax import lax
from jax.experimental import pallas as pl
from jax.experimental.pallas import tpu as pltpu
from jax.experimental.pallas import tpu_sc as plsc

N = 10000
D = 256
E = 160000
HALF = 128

NSUB = 16
NCORE = 2
CHUNK = 64
GRP_M = 8
NGRP_M = 20
GRP_D = 8
NGRP_D = 10
NBUF = 4
EP = NSUB * CHUNK * GRP_M * NGRP_M
N_PAD = 10112
ROWS_PER_SUB = N_PAD // NSUB
NDEG = 80

_mesh = plsc.VectorSubcoreMesh(core_axis_name="c", subcore_axis_name="s")


@functools.partial(
    pl.kernel,
    mesh=_mesh,
    out_type=[
        jax.ShapeDtypeStruct((NCORE * N_PAD, HALF), jnp.float32),
        jax.ShapeDtypeStruct((NCORE * NDEG, HALF), jnp.float32),
    ],
    scratch_types=[
        pltpu.VMEM_SHARED((N_PAD, HALF), jnp.float32),
        pltpu.VMEM_SHARED((NDEG, HALF), jnp.float32),
        pltpu.VMEM_SHARED((HALF, HALF), jnp.float32),
        pltpu.VMEM((2 * GRP_M, CHUNK), jnp.int32),
        pltpu.VMEM((CHUNK, HALF), jnp.float32),
        pltpu.VMEM((CHUNK, HALF), jnp.float32),
        pltpu.VMEM((CHUNK, HALF), jnp.float32),
        pltpu.VMEM((CHUNK, HALF), jnp.float32),
        pltpu.SemaphoreType.DMA,
        pltpu.SemaphoreType.DMA,
        pltpu.SemaphoreType.DMA,
        pltpu.SemaphoreType.DMA,
        pltpu.SemaphoreType.DMA,
        pltpu.SemaphoreType.DMA,
        pltpu.SemaphoreType.DMA,
        pltpu.SemaphoreType.DMA,
    ],
)
def _sc_segment_sum(x2, sd3, dd3, eye_h, zeros_h,
                    agg, deg,
                    acc, dacc, eye_s, sd_v, b0, b1, b2, b3,
                    g0, g1, g2, g3, s0, s1, s2, s3):
    cid = lax.axis_index("c")
    sid = lax.axis_index("s")
    r0 = sid * ROWS_PER_SUB

    pltpu.sync_copy(zeros_h, acc.at[pl.ds(r0, ROWS_PER_SUB)])
    pltpu.sync_copy(zeros_h.at[pl.ds(0, NDEG)], dacc)
    pltpu.sync_copy(eye_h, eye_s)
    plsc.subcore_barrier()

    bufs = [(b0, g0, s0), (b1, g1, s1), (b2, g2, s2), (b3, g3, s3)]

    def _pipelined_group(table, accum, grp, idx_row):
        pltpu.sync_copy(idx_row, sd_v)
        hg = [None] * grp
        hs = [None] * grp
        for j in range(min(NBUF - 1, grp)):
            buf, gsem, _ = bufs[j % NBUF]
            hg[j] = pltpu.async_copy(table.at[sd_v.at[j]], buf, gsem)
        for j in range(grp):
            buf, _, ssem = bufs[j % NBUF]
            hg[j].wait()
            k = j + NBUF - 1
            if k < grp:
                nbuf, ngsem, _ = bufs[k % NBUF]
                if hs[k - NBUF] is not None:
                    hs[k - NBUF].wait()
                    hs[k - NBUF] = None
                hg[k] = pltpu.async_copy(table.at[sd_v.at[k]], nbuf, ngsem)
            hs[j] = pltpu.async_copy(buf, accum.at[sd_v.at[grp + j]],
                                     ssem, add=True)
        for h in hs:
            if h is not None:
                h.wait()

    def mgroup(g, carry):
        row = (cid * NSUB + sid) * NGRP_M + g
        _pipelined_group(x2, acc, GRP_M, sd3.at[row])
        return carry

    lax.fori_loop(0, NGRP_M, mgroup, 0)

    def dgroup(g, carry):
        row = (cid * NSUB + sid) * NGRP_D + g
        _pipelined_group(eye_s, dacc, GRP_D, dd3.at[row])
        return carry

    lax.fori_loop(0, NGRP_D, dgroup, 0)
    plsc.subcore_barrier()

    pltpu.sync_copy(acc.at[pl.ds(r0, ROWS_PER_SUB)],
                    agg.at[pl.ds(cid * N_PAD + r0, ROWS_PER_SUB)])
    pltpu.sync_copy(dacc, deg.at[pl.ds(cid * NDEG, NDEG)])


def _tc_body(x_ref, a0_ref, a1_ref, d0_ref, d1_ref, ws_ref, wn0_ref, wn1_ref, b_ref, o_ref):
    deg = d0_ref[:, 0:1] + d1_ref[:, 0:1]
    r = 1.0 / jnp.maximum(deg, 1.0)
    h0 = a0_ref[...] * r
    h1 = a1_ref[...] * r
    acc = jnp.dot(x_ref[...], ws_ref[...], preferred_element_type=jnp.float32)
    acc = acc + jnp.dot(h0, wn0_ref[...], preferred_element_type=jnp.float32)
    acc = acc + jnp.dot(h1, wn1_ref[...], preferred_element_type=jnp.float32)
    o_ref[...] = acc + b_ref[0:1, :]


BLK = 632


def kernel(x, edge_index, w, Ws1, Wn1, b1, Ws2, Wn2, b2, Ws3, Wn3, b3):
    src = edge_index[0].astype(jnp.int32)
    dst = edge_index[1].astype(jnp.int32)
    pad = EP - E
    pad_src = jnp.arange(pad, dtype=jnp.int32) * 61 % N
    pad_dst = N + jnp.arange(pad, dtype=jnp.int32) % (N_PAD - N)
    src_p = jnp.concatenate([src, pad_src])
    dst_p = jnp.concatenate([dst, pad_dst])

    x2 = x.reshape(NCORE * N, HALF)

    s_r = (2 * src_p).reshape(NSUB, NGRP_M, GRP_M, CHUNK)
    d_r = dst_p.reshape(NSUB, NGRP_M, GRP_M, CHUNK)
    sd3 = jnp.concatenate([
        jnp.concatenate([s_r, d_r], axis=2)[None],
        jnp.concatenate([s_r + 1, d_r], axis=2)[None],
    ], axis=0).reshape(NCORE * NSUB * NGRP_M, 2 * GRP_M, CHUNK)

    lo_r = (dst_p & 127).reshape(NCORE, NSUB, NGRP_D, GRP_D, CHUNK)
    hi_r = (dst_p >> 7).reshape(NCORE, NSUB, NGRP_D, GRP_D, CHUNK)
    dd3 = jnp.concatenate([lo_r, hi_r], axis=3).reshape(
        NCORE * NSUB * NGRP_D, 2 * GRP_D, CHUNK)

    eye_h = jnp.eye(HALF, dtype=jnp.float32)
    zeros_h = jnp.zeros((ROWS_PER_SUB, HALF), jnp.float32)

    agg, deg2 = _sc_segment_sum(x2, sd3, dd3, eye_h, zeros_h)
    deg_a = deg2[:NDEG].reshape(NDEG * HALF)[:N]
    deg_b = deg2[NDEG:].reshape(NDEG * HALF)[:N]
    dga = jnp.broadcast_to(deg_a[:, None], (N, 16))
    dgb = jnp.broadcast_to(deg_b[:, None], (N, 16))

    wn = w / jnp.sum(w)
    Wsc = wn[0] * Ws1 + wn[1] * Ws2 + wn[2] * Ws3
    Wnc = wn[0] * Wn1 + wn[1] * Wn2 + wn[2] * Wn3
    bc = wn[0] * b1 + wn[1] * b2 + wn[2] * b3
    b_pad = jnp.zeros((8, D), jnp.float32).at[0].set(bc)

    nblk = N_PAD // BLK
    out = pl.pallas_call(
        _tc_body,
        grid=(nblk,),
        in_specs=[
            pl.BlockSpec((BLK, D), lambda i: (i, 0)),
            pl.BlockSpec((BLK, HALF), lambda i: (i, 0)),
            pl.BlockSpec((BLK, HALF), lambda i: (i + nblk, 0)),
            pl.BlockSpec((BLK, 16), lambda i: (i, 0)),
            pl.BlockSpec((BLK, 16), lambda i: (i, 0)),
            pl.BlockSpec((D, D), lambda i: (0, 0)),
            pl.BlockSpec((HALF, D), lambda i: (0, 0)),
            pl.BlockSpec((HALF, D), lambda i: (0, 0)),
            pl.BlockSpec((8, D), lambda i: (0, 0)),
        ],
        out_specs=pl.BlockSpec((BLK, D), lambda i: (i, 0)),
        out_shape=jax.ShapeDtypeStruct((N, D), jnp.float32),
    )(x, agg, agg, dga, dgb, Wsc, Wnc[:HALF], Wnc[HALF:], b_pad)
    return out


# speedup vs baseline: 1.1908x; 1.1908x over previous
import functools

import jax
import jax.numpy as jnp
from jax import lax
from jax.experimental import pallas as pl
from jax.experimental.pallas import tpu as pltpu
from jax.experimental.pallas import tpu_sc as plsc

N = 10000
D = 256
E = 160000
HALF = 128

NSUB = 16
NCORE = 2
NW = NCORE * NSUB
CHUNK = 64
GRP_M = 8
GRP_D = 4
NGRP = 20
NBUF = 4
EP = NSUB * CHUNK * GRP_M * NGRP
N_PAD = 10112
ROWS_PER_SUB = N_PAD // NSUB
NDEG = 80

_mesh = plsc.VectorSubcoreMesh(core_axis_name="c", subcore_axis_name="s")


@functools.partial(
    pl.kernel,
    mesh=_mesh,
    out_type=[
        jax.ShapeDtypeStruct((NCORE * N_PAD, HALF), jnp.float32),
        jax.ShapeDtypeStruct((NCORE * NDEG, HALF), jnp.float32),
    ],
    scratch_types=[
        pltpu.VMEM_SHARED((N_PAD, HALF), jnp.float32),
        pltpu.VMEM_SHARED((NDEG, HALF), jnp.float32),
        pltpu.VMEM((3 * GRP_M, CHUNK), jnp.int32),
        pltpu.VMEM((CHUNK, HALF), jnp.float32),
        pltpu.VMEM((CHUNK, HALF), jnp.float32),
        pltpu.VMEM((CHUNK, HALF), jnp.float32),
        pltpu.VMEM((CHUNK, HALF), jnp.float32),
        pltpu.SemaphoreType.DMA,
        pltpu.SemaphoreType.DMA,
        pltpu.SemaphoreType.DMA,
        pltpu.SemaphoreType.DMA,
        pltpu.SemaphoreType.DMA,
        pltpu.SemaphoreType.DMA,
        pltpu.SemaphoreType.DMA,
        pltpu.SemaphoreType.DMA,
    ],
)
def _sc_segment_sum(x2, eye_rep, sd3, zeros_h,
                    agg, deg,
                    acc, dacc, sd_v, b0, b1, b2, b3,
                    g0, g1, g2, g3, s0, s1, s2, s3):
    cid = lax.axis_index("c")
    sid = lax.axis_index("s")
    r0 = sid * ROWS_PER_SUB

    pltpu.sync_copy(zeros_h, acc.at[pl.ds(r0, ROWS_PER_SUB)])
    pltpu.sync_copy(zeros_h.at[pl.ds(0, NDEG)], dacc)
    plsc.subcore_barrier()

    bufs = [(b0, g0, s0), (b1, g1, s1), (b2, g2, s2), (b3, g3, s3)]

    sched = []
    for q in range(GRP_D):
        sched.append((x2, acc, 2 * q, GRP_M + 2 * q))
        sched.append((x2, acc, 2 * q + 1, GRP_M + 2 * q + 1))
        sched.append((eye_rep, dacc, 2 * GRP_M + q, 2 * GRP_M + GRP_D + q))

    def group(g, carry):
        row = (cid * NSUB + sid) * NGRP + g
        pltpu.sync_copy(sd3.at[row], sd_v)
        n = len(sched)
        hg = [None] * n
        hs = [None] * n
        for j in range(NBUF - 1):
            table, _, gi, _ = sched[j]
            buf, gsem, _ = bufs[j % NBUF]
            hg[j] = pltpu.async_copy(table.at[sd_v.at[gi]], buf, gsem)
        for j in range(n):
            _, accum, _, si = sched[j]
            buf, _, ssem = bufs[j % NBUF]
            hg[j].wait()
            k = j + NBUF - 1
            if k < n:
                ntab, _, ngi, _ = sched[k]
                nbuf, ngsem, _ = bufs[k % NBUF]
                if hs[k - NBUF] is not None:
                    hs[k - NBUF].wait()
                    hs[k - NBUF] = None
                hg[k] = pltpu.async_copy(ntab.at[sd_v.at[ngi]], nbuf, ngsem)
            hs[j] = pltpu.async_copy(buf, accum.at[sd_v.at[si]],
                                     ssem, add=True)
        for h in hs:
            if h is not None:
                h.wait()
        return carry

    lax.fori_loop(0, NGRP, group, 0)
    plsc.subcore_barrier()

    pltpu.sync_copy(acc.at[pl.ds(r0, ROWS_PER_SUB)],
                    agg.at[pl.ds(cid * N_PAD + r0, ROWS_PER_SUB)])
    pltpu.sync_copy(dacc, deg.at[pl.ds(cid * NDEG, NDEG)])


def _tc_body(x_ref, a0_ref, a1_ref, d0_ref, d1_ref, ws_ref, wn0_ref, wn1_ref, b_ref, o_ref):
    deg = d0_ref[:, 0:1] + d1_ref[:, 0:1]
    r = 1.0 / jnp.maximum(deg, 1.0)
    h0 = a0_ref[...] * r
    h1 = a1_ref[...] * r
    acc = jnp.dot(x_ref[...], ws_ref[...], preferred_element_type=jnp.float32)
    acc = acc + jnp.dot(h0, wn0_ref[...], preferred_element_type=jnp.float32)
    acc = acc + jnp.dot(h1, wn1_ref[...], preferred_element_type=jnp.float32)
    o_ref[...] = acc + b_ref[0:1, :]


BLK = 632


def kernel(x, edge_index, w, Ws1, Wn1, b1, Ws2, Wn2, b2, Ws3, Wn3, b3):
    src = edge_index[0].astype(jnp.int32)
    dst = edge_index[1].astype(jnp.int32)
    pad = EP - E
    pad_src = jnp.arange(pad, dtype=jnp.int32) * 61 % N
    pad_dst = N + jnp.arange(pad, dtype=jnp.int32) % (N_PAD - N)
    src_p = jnp.concatenate([src, pad_src])
    dst_p = jnp.concatenate([dst, pad_dst])

    x2 = x.reshape(NCORE * N, HALF)

    s_r = (2 * src_p).reshape(NSUB, NGRP, GRP_M, CHUNK)
    d_r = dst_p.reshape(NSUB, NGRP, GRP_M, CHUNK)
    wid = (jnp.arange(NCORE, dtype=jnp.int32)[:, None] * NSUB
           + jnp.arange(NSUB, dtype=jnp.int32)[None, :])
    lo_r = ((dst_p & 127).reshape(NCORE, NSUB, NGRP, GRP_D, CHUNK)
            + 128 * wid[:, :, None, None, None])
    hi_r = (dst_p >> 7).reshape(NCORE, NSUB, NGRP, GRP_D, CHUNK)
    sd3 = jnp.concatenate([
        jnp.stack([s_r, s_r + 1]),
        jnp.stack([d_r, d_r]),
        jnp.concatenate([lo_r, hi_r], axis=3),
    ], axis=3).reshape(NCORE * NSUB * NGRP, 3 * GRP_M, CHUNK)

    eye_rep = jnp.tile(jnp.eye(HALF, dtype=jnp.float32), (NW, 1))
    zeros_h = jnp.zeros((ROWS_PER_SUB, HALF), jnp.float32)

    agg, deg2 = _sc_segment_sum(x2, eye_rep, sd3, zeros_h)
    deg_a = deg2[:NDEG].reshape(NDEG * HALF)[:N]
    deg_b = deg2[NDEG:].reshape(NDEG * HALF)[:N]
    dga = jnp.broadcast_to(deg_a[:, None], (N, 16))
    dgb = jnp.broadcast_to(deg_b[:, None], (N, 16))

    wn = w / jnp.sum(w)
    Wsc = wn[0] * Ws1 + wn[1] * Ws2 + wn[2] * Ws3
    Wnc = wn[0] * Wn1 + wn[1] * Wn2 + wn[2] * Wn3
    bc = wn[0] * b1 + wn[1] * b2 + wn[2] * b3
    b_pad = jnp.zeros((8, D), jnp.float32).at[0].set(bc)

    nblk = N_PAD // BLK
    out = pl.pallas_call(
        _tc_body,
        grid=(nblk,),
        in_specs=[
            pl.BlockSpec((BLK, D), lambda i: (i, 0)),
            pl.BlockSpec((BLK, HALF), lambda i: (i, 0)),
            pl.BlockSpec((BLK, HALF), lambda i: (i + nblk, 0)),
            pl.BlockSpec((BLK, 16), lambda i: (i, 0)),
            pl.BlockSpec((BLK, 16), lambda i: (i, 0)),
            pl.BlockSpec((D, D), lambda i: (0, 0)),
            pl.BlockSpec((HALF, D), lambda i: (0, 0)),
            pl.BlockSpec((HALF, D), lambda i: (0, 0)),
            pl.BlockSpec((8, D), lambda i: (0, 0)),
        ],
        out_specs=pl.BlockSpec((BLK, D), lambda i: (i, 0)),
        out_shape=jax.ShapeDtypeStruct((N, D), jnp.float32),
    )(x, agg, agg, dga, dgb, Wsc, Wnc[:HALF], Wnc[HALF:], b_pad)
    return out
